# baseline (speedup 1.0000x reference)
import functools

import jax
import jax.numpy as jnp
from jax import lax
from jax.experimental import pallas as pl
from jax.experimental.pallas import tpu as pltpu
from jax.experimental.pallas import tpu_sc as plsc

_VOCAB = 1000001
_HID = 64
_LABELS = 16
_BATCH = 4096
_SEQ = 200
_CHUNK = 100
_NCHUNK = _SEQ // _CHUNK
_NBUF = 16
_UNROLL = 4
_VBLK = 32768


def _pool_kernel(ids_hbm, p_hbm, b_hbm, out_hbm, idx_v, rows_v, out_v,
                 bias_v, *sems):
    info = plsc.get_sparse_core_info()
    nw = info.num_cores * info.num_subcores
    rows_per_w = _BATCH // nw
    wid = lax.axis_index("s") * info.num_cores + lax.axis_index("c")
    base = wid * rows_per_w

    pltpu.sync_copy(ids_hbm.at[pl.ds(base, rows_per_w)], idx_v)
    pltpu.sync_copy(b_hbm, bias_v)

    def issue(row, buf):
        for c in range(_NCHUNK):
            pltpu.async_copy(
                p_hbm.at[idx_v.at[row, c]], rows_v.at[buf, c], sems[buf])

    def wait(row, buf):
        for c in range(_NCHUNK):
            pltpu.make_async_copy(
                p_hbm.at[idx_v.at[row, c]], rows_v.at[buf, c],
                sems[buf]).wait()

    for b in range(_NBUF):
        issue(b, b)

    inv = jnp.full((16,), 1.0 / _SEQ, dtype=jnp.float32)
    bias = bias_v[...]

    def outer(i, _):
        for b in range(_NBUF):
            row = i * _NBUF + b
            wait(row, b)

            def accum(c):
                def body(it, accs):
                    rr = it * _UNROLL
                    return tuple(
                        accs[u] + rows_v[b, c, rr + u]
                        for u in range(_UNROLL))
                return body

            accs = tuple(jnp.zeros((16,), jnp.float32) for _ in range(_UNROLL))
            for c in range(_NCHUNK):
                accs = lax.fori_loop(0, _CHUNK // _UNROLL, accum(c), accs)

            @pl.when(row + _NBUF < rows_per_w)
            def _():
                issue(row + _NBUF, b)

            total = (accs[0] + accs[1]) + (accs[2] + accs[3])
            out_v[row] = total * inv + bias
        return 0

    lax.fori_loop(0, rows_per_w // _NBUF, outer, 0)
    pltpu.sync_copy(out_v, out_hbm.at[pl.ds(base, rows_per_w)])


def _make_pool():
    info = plsc.get_sparse_core_info()
    nw = info.num_cores * info.num_subcores
    rows_per_w = _BATCH // nw
    mesh = plsc.VectorSubcoreMesh(core_axis_name="c", subcore_axis_name="s")
    return pl.kernel(
        _pool_kernel,
        out_type=jax.ShapeDtypeStruct((_BATCH, _LABELS), jnp.float32),
        mesh=mesh,
        scratch_types=[
            pltpu.VMEM((rows_per_w, _NCHUNK, _CHUNK), jnp.int32),
            pltpu.VMEM((_NBUF, _NCHUNK, _CHUNK, _LABELS), jnp.float32),
            pltpu.VMEM((rows_per_w, _LABELS), jnp.float32),
            pltpu.VMEM((_LABELS,), jnp.float32),
        ] + [pltpu.SemaphoreType.DMA] * _NBUF,
        compiler_params=pltpu.CompilerParams(use_tc_tiling_on_sc=False),
    )


def _matmul_kernel(embt_ref, w_ref, p_ref):
    g = _VBLK // 8
    acc = lax.dot_general(
        embt_ref[:, :g], w_ref[:_HID],
        (((0,), (0,)), ((), ())), preferred_element_type=jnp.float32)
    for j in range(1, 8):
        acc = acc + lax.dot_general(
            embt_ref[:, j * g:(j + 1) * g], w_ref[j * _HID:(j + 1) * _HID],
            (((0,), (0,)), ((), ())), preferred_element_type=jnp.float32)
    p_ref[...] = acc


_NBLK = (_VOCAB + _VBLK - 1) // _VBLK
_VPAD = _NBLK * _VBLK
_GBLK = _VBLK // 8


def _project_table(emb, W):
    embt = emb.T
    wbig = jnp.zeros((8 * _HID, 128), jnp.float32)
    for j in range(8):
        wbig = wbig.at[j * _HID:(j + 1) * _HID, 16 * j:16 * (j + 1)].set(W.T)
    packed = pl.pallas_call(
        _matmul_kernel,
        grid=(_NBLK,),
        in_specs=[
            pl.BlockSpec((_HID, _VBLK), lambda i: (0, i)),
            pl.BlockSpec((8 * _HID, 128), lambda i: (0, 0)),
        ],
        out_specs=pl.BlockSpec((_VBLK // 8, 128), lambda i: (i, 0)),
        out_shape=jax.ShapeDtypeStruct((_VPAD // 8, 128), jnp.float32),
    )(embt, wbig)
    return packed.reshape(_VPAD, _LABELS)


def kernel(input_ids, emb, W, b):
    ids = input_ids.astype(jnp.int32)
    ids = ((ids & ~(_VBLK - 1))
           | ((ids & (_GBLK - 1)) << 3)
           | ((ids >> (_GBLK.bit_length() - 1)) & 7))
    ids = ids.reshape(_BATCH, _NCHUNK, _CHUNK)
    p = _project_table(emb, W)
    return _make_pool()(ids, p, b)


# speedup vs baseline: 1.0012x; 1.0012x over previous
import functools

import jax
import jax.numpy as jnp
from jax import lax
from jax.experimental import pallas as pl
from jax.experimental.pallas import tpu as pltpu
from jax.experimental.pallas import tpu_sc as plsc

_VOCAB = 1000001
_HID = 64
_LABELS = 16
_BATCH = 4096
_SEQ = 200
_CHUNK = 100
_NCHUNK = _SEQ // _CHUNK
_NBUF = 8
_UNROLL = 10
_VBLK = 32768


def _pool_kernel(ids_hbm, p_hbm, b_hbm, out_hbm, idx_v, rows_v, out_v,
                 bias_v, *sems):
    info = plsc.get_sparse_core_info()
    nw = info.num_cores * info.num_subcores
    rows_per_w = _BATCH // nw
    wid = lax.axis_index("s") * info.num_cores + lax.axis_index("c")
    base = wid * rows_per_w

    pltpu.sync_copy(ids_hbm.at[pl.ds(base, rows_per_w)], idx_v)
    pltpu.sync_copy(b_hbm, bias_v)

    def issue(row, buf):
        for c in range(_NCHUNK):
            pltpu.async_copy(
                p_hbm.at[idx_v.at[row, c]], rows_v.at[buf, c], sems[buf])

    def wait(row, buf):
        for c in range(_NCHUNK):
            pltpu.make_async_copy(
                p_hbm.at[idx_v.at[row, c]], rows_v.at[buf, c],
                sems[buf]).wait()

    for b in range(_NBUF):
        issue(b, b)

    inv = jnp.full((16,), 1.0 / _SEQ, dtype=jnp.float32)
    bias = bias_v[...]

    def outer(i, _):
        for b in range(_NBUF):
            row = i * _NBUF + b
            wait(row, b)

            def accum(c):
                def body(it, accs):
                    rr = it * _UNROLL
                    return tuple(
                        accs[u] + rows_v[b, c, rr + u]
                        for u in range(_UNROLL))
                return body

            accs = tuple(jnp.zeros((16,), jnp.float32) for _ in range(_UNROLL))
            for c in range(_NCHUNK):
                accs = lax.fori_loop(0, _CHUNK // _UNROLL, accum(c), accs)

            @pl.when(row + _NBUF < rows_per_w)
            def _():
                issue(row + _NBUF, b)

            half = _UNROLL // 2
            pairs = [accs[2 * q] + accs[2 * q + 1] for q in range(half)]
            while len(pairs) > 1:
                pairs = [pairs[2 * q] + pairs[2 * q + 1]
                         for q in range(len(pairs) // 2)] + (
                             [pairs[-1]] if len(pairs) % 2 else [])
            total = pairs[0]
            out_v[row] = total * inv + bias
        return 0

    lax.fori_loop(0, rows_per_w // _NBUF, outer, 0)
    pltpu.sync_copy(out_v, out_hbm.at[pl.ds(base, rows_per_w)])


def _make_pool():
    info = plsc.get_sparse_core_info()
    nw = info.num_cores * info.num_subcores
    rows_per_w = _BATCH // nw
    mesh = plsc.VectorSubcoreMesh(core_axis_name="c", subcore_axis_name="s")
    return pl.kernel(
        _pool_kernel,
        out_type=jax.ShapeDtypeStruct((_BATCH, _LABELS), jnp.float32),
        mesh=mesh,
        scratch_types=[
            pltpu.VMEM((rows_per_w, _NCHUNK, _CHUNK), jnp.int32),
            pltpu.VMEM((_NBUF, _NCHUNK, _CHUNK, _LABELS), jnp.float32),
            pltpu.VMEM((rows_per_w, _LABELS), jnp.float32),
            pltpu.VMEM((_LABELS,), jnp.float32),
        ] + [pltpu.SemaphoreType.DMA] * _NBUF,
        compiler_params=pltpu.CompilerParams(use_tc_tiling_on_sc=False),
    )


def _matmul_kernel(embt_ref, w_ref, p_ref):
    g = _VBLK // 8
    acc = lax.dot_general(
        embt_ref[:, :g], w_ref[:_HID],
        (((0,), (0,)), ((), ())), preferred_element_type=jnp.float32)
    for j in range(1, 8):
        acc = acc + lax.dot_general(
            embt_ref[:, j * g:(j + 1) * g], w_ref[j * _HID:(j + 1) * _HID],
            (((0,), (0,)), ((), ())), preferred_element_type=jnp.float32)
    p_ref[...] = acc


_NBLK = (_VOCAB + _VBLK - 1) // _VBLK
_VPAD = _NBLK * _VBLK
_GBLK = _VBLK // 8


def _project_table(emb, W):
    embt = emb.T
    wbig = jnp.zeros((8 * _HID, 128), jnp.float32)
    for j in range(8):
        wbig = wbig.at[j * _HID:(j + 1) * _HID, 16 * j:16 * (j + 1)].set(W.T)
    packed = pl.pallas_call(
        _matmul_kernel,
        grid=(_NBLK,),
        in_specs=[
            pl.BlockSpec((_HID, _VBLK), lambda i: (0, i)),
            pl.BlockSpec((8 * _HID, 128), lambda i: (0, 0)),
        ],
        out_specs=pl.BlockSpec((_VBLK // 8, 128), lambda i: (i, 0)),
        out_shape=jax.ShapeDtypeStruct((_VPAD // 8, 128), jnp.float32),
    )(embt, wbig)
    return packed.reshape(_VPAD, _LABELS)


def kernel(input_ids, emb, W, b):
    ids = input_ids.astype(jnp.int32)
    ids = ((ids & ~(_VBLK - 1))
           | ((ids & (_GBLK - 1)) << 3)
           | ((ids >> (_GBLK.bit_length() - 1)) & 7))
    ids = ids.reshape(_BATCH, _NCHUNK, _CHUNK)
    p = _project_table(emb, W)
    return _make_pool()(ids, p, b)


# speedup vs baseline: 1.0032x; 1.0020x over previous
import jax
import jax.numpy as jnp
from jax import lax
from jax.experimental import pallas as pl
from jax.experimental.pallas import tpu as pltpu
from jax.experimental.pallas import tpu_sc as plsc

_VOCAB = 1000001
_HID = 64
_LABELS = 16
_BATCH = 4096
_SEQ = 200
_CHUNK = 100
_NCHUNK = _SEQ // _CHUNK
_NBUF = 8
_UNROLL = 10
_VBLK = 32768


def _pool_kernel(ids_hbm, p_hbm, b_hbm, out_hbm, idx_v, rows_v, out_v,
                 bias_v, *sems):
    info = plsc.get_sparse_core_info()
    nw = info.num_cores * info.num_subcores
    rows_per_w = _BATCH // nw
    wid = lax.axis_index("s") * info.num_cores + lax.axis_index("c")
    base = wid * rows_per_w

    pltpu.sync_copy(ids_hbm.at[pl.ds(base, rows_per_w)], idx_v)
    pltpu.sync_copy(b_hbm, bias_v)

    def issue(row, buf):
        for c in range(_NCHUNK):
            pltpu.async_copy(
                p_hbm.at[idx_v.at[row, c]], rows_v.at[buf, c], sems[buf])

    def wait(row, buf):
        for c in range(_NCHUNK):
            pltpu.make_async_copy(
                p_hbm.at[idx_v.at[row, c]], rows_v.at[buf, c],
                sems[buf]).wait()

    for b in range(_NBUF):
        issue(b, b)

    inv = jnp.full((16,), 1.0 / _SEQ, dtype=jnp.float32)
    bias = bias_v[...]

    def outer(i, _):
        for b in range(_NBUF):
            row = i * _NBUF + b
            wait(row, b)

            def accum(c):
                def body(it, accs):
                    rr = it * _UNROLL
                    return tuple(
                        accs[u] + rows_v[b, c, rr + u]
                        for u in range(_UNROLL))
                return body

            accs = tuple(jnp.zeros((16,), jnp.float32) for _ in range(_UNROLL))
            for c in range(_NCHUNK):
                accs = lax.fori_loop(0, _CHUNK // _UNROLL, accum(c), accs)

            @pl.when(row + _NBUF < rows_per_w)
            def _():
                issue(row + _NBUF, b)

            half = _UNROLL // 2
            pairs = [accs[2 * q] + accs[2 * q + 1] for q in range(half)]
            while len(pairs) > 1:
                pairs = [pairs[2 * q] + pairs[2 * q + 1]
                         for q in range(len(pairs) // 2)] + (
                             [pairs[-1]] if len(pairs) % 2 else [])
            total = pairs[0]
            out_v[row] = total * inv + bias
        return 0

    lax.fori_loop(0, rows_per_w // _NBUF, outer, 0)
    pltpu.sync_copy(out_v, out_hbm.at[pl.ds(base, rows_per_w)])


def _make_pool():
    info = plsc.get_sparse_core_info()
    nw = info.num_cores * info.num_subcores
    rows_per_w = _BATCH // nw
    mesh = plsc.VectorSubcoreMesh(core_axis_name="c", subcore_axis_name="s")
    return pl.kernel(
        _pool_kernel,
        out_type=jax.ShapeDtypeStruct((_BATCH, _LABELS), jnp.float32),
        mesh=mesh,
        scratch_types=[
            pltpu.VMEM((rows_per_w, _NCHUNK, _CHUNK), jnp.int32),
            pltpu.VMEM((_NBUF, _NCHUNK, _CHUNK, _LABELS), jnp.float32),
            pltpu.VMEM((rows_per_w, _LABELS), jnp.float32),
            pltpu.VMEM((_LABELS,), jnp.float32),
        ] + [pltpu.SemaphoreType.DMA] * _NBUF,
        compiler_params=pltpu.CompilerParams(use_tc_tiling_on_sc=False),
    )


def _matmul_kernel(embt_ref, w_ref, p_ref):
    g = _VBLK // 8
    acc = lax.dot_general(
        embt_ref[:, :g], w_ref[:_HID],
        (((0,), (0,)), ((), ())), preferred_element_type=jnp.float32)
    for j in range(1, 8):
        acc = acc + lax.dot_general(
            embt_ref[:, j * g:(j + 1) * g], w_ref[j * _HID:(j + 1) * _HID],
            (((0,), (0,)), ((), ())), preferred_element_type=jnp.float32)
    p_ref[...] = acc


_NBLK = (_VOCAB + _VBLK - 1) // _VBLK
_VPAD = _NBLK * _VBLK
_GBLK = _VBLK // 8


def _project_table(emb, W):
    embt = emb.T
    wbig = jnp.zeros((8 * _HID, 128), jnp.float32)
    for j in range(8):
        wbig = wbig.at[j * _HID:(j + 1) * _HID, 16 * j:16 * (j + 1)].set(W.T)
    packed = pl.pallas_call(
        _matmul_kernel,
        grid=(_NBLK,),
        in_specs=[
            pl.BlockSpec((_HID, _VBLK), lambda i: (0, i)),
            pl.BlockSpec((8 * _HID, 128), lambda i: (0, 0)),
        ],
        out_specs=pl.BlockSpec((_VBLK // 8, 128), lambda i: (i, 0)),
        out_shape=jax.ShapeDtypeStruct((_VPAD // 8, 128), jnp.float32),
    )(embt, wbig)
    return packed.reshape(_VPAD, _LABELS)


def kernel(input_ids, emb, W, b):
    ids = input_ids.astype(jnp.int32)
    ids = ((ids & ~(_VBLK - 1))
           | ((ids & (_GBLK - 1)) << 3)
           | ((ids >> (_GBLK.bit_length() - 1)) & 7))
    ids = ids.reshape(_BATCH, _NCHUNK, _CHUNK)
    p = _project_table(emb, W)
    return _make_pool()(ids, p, b)
